# baseline (device time: 225811 ns/iter reference)
import jax
import jax.numpy as jnp
from jax import lax
from jax.experimental import pallas as pl
from jax.experimental.pallas import tpu as pltpu

N_DEV = 4
H_LOC = 8
DH = 128
SQ = 1024
SKV_LOC = 1024
SKV = N_DEV * SKV_LOC
SCALE = 0.08838834764831843
HD = H_LOC * DH


def _body(x_ref, wq_ref, wo_ref, kt_ref, vt_ref, out_ref,
          kbuf, vbuf, qbuf, ar_send, ar_recv,
          send_a2a, recv_a2a, local_sem, send_ar, recv_ar):
    me = lax.axis_index("i")

    barrier_sem = pltpu.get_barrier_semaphore()
    for o in range(1, N_DEV):
        pl.semaphore_signal(
            barrier_sem, inc=1,
            device_id=(lax.rem(me + o, N_DEV),),
            device_id_type=pl.DeviceIdType.MESH,
        )
    pl.semaphore_wait(barrier_sem, N_DEV - 1)

    a2a_sends = []
    for o in range(1, N_DEV):
        d = lax.rem(me + o, N_DEV)
        for t, (src, dst) in enumerate(((kt_ref, kbuf), (vt_ref, vbuf))):
            rdma = pltpu.make_async_remote_copy(
                src_ref=src.at[:, pl.ds(d * HD, HD)],
                dst_ref=dst.at[pl.ds(me * SKV_LOC, SKV_LOC), :],
                send_sem=send_a2a.at[o - 1, t],
                recv_sem=recv_a2a.at[3 - o, t],
                device_id=(d,),
                device_id_type=pl.DeviceIdType.MESH,
            )
            rdma.start()
            a2a_sends.append(rdma)

    local_copies = []
    for t, (src, dst) in enumerate(((kt_ref, kbuf), (vt_ref, vbuf))):
        cp = pltpu.make_async_copy(
            src.at[:, pl.ds(me * HD, HD)],
            dst.at[pl.ds(me * SKV_LOC, SKV_LOC), :],
            local_sem.at[t],
        )
        cp.start()
        local_copies.append(cp)

    q = lax.dot_general(
        x_ref[...], wq_ref[...],
        (((1,), (0,)), ((), ())),
        preferred_element_type=jnp.float32,
    )
    qbuf[...] = (q * SCALE).astype(jnp.bfloat16)

    for cp in local_copies:
        cp.wait()
    for oo in range(1, N_DEV):
        j = lax.rem(me + oo, N_DEV)
        for t, dst in enumerate((kbuf, vbuf)):
            pltpu.make_async_remote_copy(
                src_ref=kt_ref.at[:, pl.ds(0, HD)],
                dst_ref=dst.at[pl.ds(j * SKV_LOC, SKV_LOC), :],
                send_sem=send_a2a.at[0, 0],
                recv_sem=recv_a2a.at[oo - 1, t],
                device_id=(me,),
                device_id_type=pl.DeviceIdType.MESH,
            ).wait_recv()

    for h in range(H_LOC):
        qh = qbuf[:, h * DH:(h + 1) * DH]
        kh = kbuf[:, h * DH:(h + 1) * DH]
        vh = vbuf[:, h * DH:(h + 1) * DH]
        woh = wo_ref[h * DH:(h + 1) * DH, :]
        for c in range(4):
            qc = qh.reshape(4, 4, 64, DH)[:, c].reshape(256, DH)
            kc = kh.reshape(16, 4, 64, DH)[:, c].reshape(1024, DH)
            vc = vh.reshape(16, 4, 64, DH)[:, c].reshape(1024, DH)
            s = lax.dot_general(
                qc, kc, (((1,), (1,)), ((), ())),
                preferred_element_type=jnp.float32,
            )
            m = jnp.max(s, axis=1, keepdims=True)
            p = jnp.exp(s - m)
            l = jnp.sum(p, axis=1, keepdims=True)
            pn = (p / l).astype(jnp.bfloat16)
            ctx = lax.dot_general(
                pn, vc, (((1,), (0,)), ((), ())),
                preferred_element_type=jnp.float32,
            )
            upd = lax.dot_general(
                ctx.astype(jnp.bfloat16), woh,
                (((1,), (0,)), ((), ())),
                preferred_element_type=jnp.float32,
            )
            for mm in range(4):
                base = 256 * mm + 64 * c
                blk = upd[64 * mm:64 * (mm + 1), :]
                if h == 0:
                    out_ref[0, base:base + 64, :] = blk
                else:
                    out_ref[0, base:base + 64, :] = (
                        out_ref[0, base:base + 64, :] + blk
                    )

    ar_send[...] = out_ref[0].astype(jnp.bfloat16)
    ar_sends = []
    for o in range(1, N_DEV):
        d = lax.rem(me + o, N_DEV)
        rdma = pltpu.make_async_remote_copy(
            src_ref=ar_send,
            dst_ref=ar_recv.at[3 - o],
            send_sem=send_ar.at[o - 1],
            recv_sem=recv_ar.at[3 - o],
            device_id=(d,),
            device_id_type=pl.DeviceIdType.MESH,
        )
        rdma.start()
        ar_sends.append(rdma)

    for s_ in range(N_DEV - 1):
        pltpu.make_async_remote_copy(
            src_ref=ar_send,
            dst_ref=ar_recv.at[s_],
            send_sem=send_ar.at[0],
            recv_sem=recv_ar.at[s_],
            device_id=(me,),
            device_id_type=pl.DeviceIdType.MESH,
        ).wait_recv()

    total = out_ref[0]
    for s_ in range(N_DEV - 1):
        total = total + ar_recv[s_].astype(jnp.float32)
    out_ref[0] = total

    for rdma in a2a_sends:
        rdma.wait_send()
    for rdma in ar_sends:
        rdma.wait_send()


def kernel(x, Wq, K_ext, V_ext, Wo):
    xb = x[0].astype(jnp.bfloat16)
    wqb = Wq.astype(jnp.bfloat16)
    wob = Wo.astype(jnp.bfloat16)
    kt = K_ext[0].astype(jnp.bfloat16).reshape(SKV_LOC, N_DEV * HD)
    vt = V_ext[0].astype(jnp.bfloat16).reshape(SKV_LOC, N_DEV * HD)

    return pl.pallas_call(
        _body,
        out_shape=jax.ShapeDtypeStruct((1, SQ, 1024), jnp.float32),
        in_specs=[pl.BlockSpec(memory_space=pltpu.VMEM)] * 5,
        out_specs=pl.BlockSpec(memory_space=pltpu.VMEM),
        scratch_shapes=[
            pltpu.VMEM((SKV, HD), jnp.bfloat16),
            pltpu.VMEM((SKV, HD), jnp.bfloat16),
            pltpu.VMEM((SQ, HD), jnp.bfloat16),
            pltpu.VMEM((SQ, 1024), jnp.bfloat16),
            pltpu.VMEM((3, SQ, 1024), jnp.bfloat16),
            pltpu.SemaphoreType.DMA((3, 2)),
            pltpu.SemaphoreType.DMA((3, 2)),
            pltpu.SemaphoreType.DMA((2,)),
            pltpu.SemaphoreType.DMA((3,)),
            pltpu.SemaphoreType.DMA((3,)),
        ],
        compiler_params=pltpu.CompilerParams(
            collective_id=0, vmem_limit_bytes=100 * 1024 * 1024,
        ),
    )(xb, wqb, wob, kt, vt)


# device time: 217822 ns/iter; 1.0367x vs baseline; 1.0367x over previous
import jax
import jax.numpy as jnp
from jax import lax
from jax.experimental import pallas as pl
from jax.experimental.pallas import tpu as pltpu

N_DEV = 4
H_LOC = 8
DH = 128
SQ = 1024
SKV_LOC = 1024
SKV = N_DEV * SKV_LOC
SCALE = 0.08838834764831843
HD = H_LOC * DH


def _body(x_ref, wq_ref, wo_ref, kt_ref, vt_ref, out_ref,
          kbuf, vbuf, qbuf, ar_send, ar_recv,
          send_a2a, recv_a2a, local_sem, send_ar, recv_ar):
    me = lax.axis_index("i")

    barrier_sem = pltpu.get_barrier_semaphore()
    for o in range(1, N_DEV):
        pl.semaphore_signal(
            barrier_sem, inc=1,
            device_id=(lax.rem(me + o, N_DEV),),
            device_id_type=pl.DeviceIdType.MESH,
        )
    pl.semaphore_wait(barrier_sem, N_DEV - 1)

    a2a_sends = []
    for o in range(1, N_DEV):
        d = lax.rem(me + o, N_DEV)
        for t, (src, dst) in enumerate(((kt_ref, kbuf), (vt_ref, vbuf))):
            rdma = pltpu.make_async_remote_copy(
                src_ref=src.at[:, pl.ds(d * HD, HD)],
                dst_ref=dst.at[pl.ds(me * SKV_LOC, SKV_LOC), :],
                send_sem=send_a2a.at[o - 1, t],
                recv_sem=recv_a2a.at[3 - o, t],
                device_id=(d,),
                device_id_type=pl.DeviceIdType.MESH,
            )
            rdma.start()
            a2a_sends.append(rdma)

    local_copies = []
    for t, (src, dst) in enumerate(((kt_ref, kbuf), (vt_ref, vbuf))):
        cp = pltpu.make_async_copy(
            src.at[:, pl.ds(me * HD, HD)],
            dst.at[pl.ds(me * SKV_LOC, SKV_LOC), :],
            local_sem.at[t],
        )
        cp.start()
        local_copies.append(cp)

    q = lax.dot_general(
        x_ref[...], wq_ref[...],
        (((1,), (0,)), ((), ())),
        preferred_element_type=jnp.float32,
    )
    qbuf[...] = (q * SCALE).astype(jnp.bfloat16)

    for cp in local_copies:
        cp.wait()
    for oo in range(1, N_DEV):
        j = lax.rem(me + oo, N_DEV)
        for t, dst in enumerate((kbuf, vbuf)):
            pltpu.make_async_remote_copy(
                src_ref=kt_ref.at[:, pl.ds(0, HD)],
                dst_ref=dst.at[pl.ds(j * SKV_LOC, SKV_LOC), :],
                send_sem=send_a2a.at[0, 0],
                recv_sem=recv_a2a.at[oo - 1, t],
                device_id=(me,),
                device_id_type=pl.DeviceIdType.MESH,
            ).wait_recv()

    ar_sends = []
    for half in range(2):
        for h in range(H_LOC):
            qh = qbuf[:, h * DH:(h + 1) * DH]
            kh = kbuf[:, h * DH:(h + 1) * DH]
            vh = vbuf[:, h * DH:(h + 1) * DH]
            woh = wo_ref[h * DH:(h + 1) * DH, :]
            for c in range(4):
                qc = (qh.reshape(4, 4, 64, DH)[2 * half:2 * half + 2, c]
                      .reshape(128, DH))
                kc = kh.reshape(16, 4, 64, DH)[:, c].reshape(1024, DH)
                vc = vh.reshape(16, 4, 64, DH)[:, c].reshape(1024, DH)
                s = lax.dot_general(
                    qc, kc, (((1,), (1,)), ((), ())),
                    preferred_element_type=jnp.float32,
                )
                m = jnp.max(s, axis=1, keepdims=True)
                p = jnp.exp(s - m)
                l = jnp.sum(p, axis=1, keepdims=True)
                pn = (p / l).astype(jnp.bfloat16)
                ctx = lax.dot_general(
                    pn, vc, (((1,), (0,)), ((), ())),
                    preferred_element_type=jnp.float32,
                )
                upd = lax.dot_general(
                    ctx.astype(jnp.bfloat16), woh,
                    (((1,), (0,)), ((), ())),
                    preferred_element_type=jnp.float32,
                )
                for mm in range(2):
                    base = 256 * (2 * half + mm) + 64 * c
                    blk = upd[64 * mm:64 * (mm + 1), :]
                    if h == 0:
                        out_ref[0, base:base + 64, :] = blk
                    else:
                        out_ref[0, base:base + 64, :] = (
                            out_ref[0, base:base + 64, :] + blk
                        )

        rows = slice(512 * half, 512 * (half + 1))
        ar_send[rows, :] = out_ref[0, rows, :].astype(jnp.bfloat16)
        for o in range(1, N_DEV):
            d = lax.rem(me + o, N_DEV)
            rdma = pltpu.make_async_remote_copy(
                src_ref=ar_send.at[rows, :],
                dst_ref=ar_recv.at[3 - o, rows, :],
                send_sem=send_ar.at[half, o - 1],
                recv_sem=recv_ar.at[half, 3 - o],
                device_id=(d,),
                device_id_type=pl.DeviceIdType.MESH,
            )
            rdma.start()
            ar_sends.append(rdma)

    for half in range(2):
        rows = slice(512 * half, 512 * (half + 1))
        for s_ in range(N_DEV - 1):
            pltpu.make_async_remote_copy(
                src_ref=ar_send.at[rows, :],
                dst_ref=ar_recv.at[s_, rows, :],
                send_sem=send_ar.at[0, 0],
                recv_sem=recv_ar.at[half, s_],
                device_id=(me,),
                device_id_type=pl.DeviceIdType.MESH,
            ).wait_recv()
        total = out_ref[0, rows, :]
        for s_ in range(N_DEV - 1):
            total = total + ar_recv[s_, rows, :].astype(jnp.float32)
        out_ref[0, rows, :] = total

    for rdma in a2a_sends:
        rdma.wait_send()
    for rdma in ar_sends:
        rdma.wait_send()


def kernel(x, Wq, K_ext, V_ext, Wo):
    xb = x[0].astype(jnp.bfloat16)
    wqb = Wq.astype(jnp.bfloat16)
    wob = Wo.astype(jnp.bfloat16)
    kt = K_ext[0].astype(jnp.bfloat16).reshape(SKV_LOC, N_DEV * HD)
    vt = V_ext[0].astype(jnp.bfloat16).reshape(SKV_LOC, N_DEV * HD)

    return pl.pallas_call(
        _body,
        out_shape=jax.ShapeDtypeStruct((1, SQ, 1024), jnp.float32),
        in_specs=[pl.BlockSpec(memory_space=pltpu.VMEM)] * 5,
        out_specs=pl.BlockSpec(memory_space=pltpu.VMEM),
        scratch_shapes=[
            pltpu.VMEM((SKV, HD), jnp.bfloat16),
            pltpu.VMEM((SKV, HD), jnp.bfloat16),
            pltpu.VMEM((SQ, HD), jnp.bfloat16),
            pltpu.VMEM((SQ, 1024), jnp.bfloat16),
            pltpu.VMEM((3, SQ, 1024), jnp.bfloat16),
            pltpu.SemaphoreType.DMA((3, 2)),
            pltpu.SemaphoreType.DMA((3, 2)),
            pltpu.SemaphoreType.DMA((2,)),
            pltpu.SemaphoreType.DMA((2, 3)),
            pltpu.SemaphoreType.DMA((2, 3)),
        ],
        compiler_params=pltpu.CompilerParams(
            collective_id=0, vmem_limit_bytes=100 * 1024 * 1024,
        ),
    )(xb, wqb, wob, kt, vt)
